# baseline (speedup 1.0000x reference)
import functools

import jax
import jax.numpy as jnp
from jax import lax
from jax.experimental import pallas as pl
from jax.experimental.pallas import tpu as pltpu
from jax.experimental.pallas import tpu_sc as plsc

N = 20000
B = 16
NOBJ = 32
C = 81
THRESHOLD = 0.5
NEG_POS = 3
AS, AL = 160, 125
CHUNK, NCHUNK = 800, 25

_INTERPRET = False


def _match_kernel(db_ref, boxes_ref, labels_ref, locs_ref, tcls_ref, stats_ref):
    dcx = db_ref[0]
    dcy = db_ref[1]
    dw = db_ref[2]
    dh = db_ref[3]
    dx0 = dcx - dw / 2.0
    dy0 = dcy - dh / 2.0
    dx1 = dcx + dw / 2.0
    dy1 = dcy + dh / 2.0
    darea = (dx1 - dx0) * (dy1 - dy0)

    row_ids = jax.lax.broadcasted_iota(jnp.int32, (AS, AL), 0)
    col_ids = jax.lax.broadcasted_iota(jnp.int32, (AS, AL), 1)
    aidx = row_ids * AL + col_ids

    best = jnp.full((AS, AL), -1.0, jnp.float32)
    besti = jnp.zeros((AS, AL), jnp.int32)
    dbj = []
    for j in range(NOBJ):
        bx0 = boxes_ref[0, 0, j]
        by0 = boxes_ref[0, 1, j]
        bx1 = boxes_ref[0, 2, j]
        by1 = boxes_ref[0, 3, j]
        barea = (bx1 - bx0) * (by1 - by0)
        ix = jnp.maximum(jnp.minimum(bx1, dx1) - jnp.maximum(bx0, dx0), 0.0)
        iy = jnp.maximum(jnp.minimum(by1, dy1) - jnp.maximum(by0, dy0), 0.0)
        inter = ix * iy
        union = jnp.maximum(barea + darea - inter, 1e-10)
        iou = inter / union
        upd = iou > best
        besti = jnp.where(upd, j, besti)
        best = jnp.where(upd, iou, best)
        mx = jnp.max(iou)
        am = jnp.min(jnp.where(iou == mx, aidx, N))
        dbj.append(am)

    fj = jnp.full((AS, AL), -1, jnp.int32)
    for j in range(NOBJ):
        fj = jnp.where(aidx == dbj[j], j, fj)
    o = jnp.where(fj >= 0, fj, besti)
    ovl = jnp.where(fj >= 0, 1.0, best)

    lab = jnp.zeros((AS, AL), jnp.int32)
    gx0 = jnp.zeros((AS, AL), jnp.float32)
    gy0 = jnp.zeros((AS, AL), jnp.float32)
    gx1 = jnp.zeros((AS, AL), jnp.float32)
    gy1 = jnp.zeros((AS, AL), jnp.float32)
    for j in range(NOBJ):
        m = o == j
        lab = jnp.where(m, labels_ref[0, 0, j], lab)
        gx0 = jnp.where(m, boxes_ref[0, 0, j], gx0)
        gy0 = jnp.where(m, boxes_ref[0, 1, j], gy0)
        gx1 = jnp.where(m, boxes_ref[0, 2, j], gx1)
        gy1 = jnp.where(m, boxes_ref[0, 3, j], gy1)
    lab = jnp.where(ovl < THRESHOLD, 0, lab)
    tcls_ref[0] = lab

    pos = lab != 0
    n_pos = jnp.sum(pos.astype(jnp.float32))

    cxt = (gx0 + gx1) / 2.0
    cyt = (gy0 + gy1) / 2.0
    wt = gx1 - gx0
    ht = gy1 - gy0
    g0 = (cxt - dcx) / (dw / 10.0)
    g1 = (cyt - dcy) / (dh / 10.0)
    g2 = jnp.log(jnp.maximum(wt / dw, 1e-8)) * 5.0
    g3 = jnp.log(jnp.maximum(ht / dh, 1e-8)) * 5.0
    sl = jnp.zeros((AS, AL), jnp.float32)
    for c, g in enumerate((g0, g1, g2, g3)):
        ad = jnp.abs(locs_ref[0, c] - g)
        sl = sl + jnp.where(ad < 1.0, 0.5 * ad * ad, ad - 0.5)
    loc_num = jnp.sum(jnp.where(pos, sl, 0.0))

    lane = jax.lax.broadcasted_iota(jnp.int32, (1, 128), 1)
    stats_ref[0] = jnp.where(lane == 0, n_pos, jnp.where(lane == 1, loc_num, 0.0))


def _ce_kernel(cls_ref, lab_ref, neg_ref, stats_ref):
    cls_iota = jax.lax.broadcasted_iota(jnp.int32, (CHUNK, C), 1)
    lane_nc = jax.lax.broadcasted_iota(jnp.int32, (CHUNK, NCHUNK), 1)
    labfull = lab_ref[0]

    def body(i, carry):
        acc, negacc = carry
        x = cls_ref[0, pl.ds(i * CHUNK, CHUNK), :]
        lab = jnp.sum(jnp.where(lane_nc == i, labfull, 0), axis=1, keepdims=True)
        s = jnp.sum(jnp.exp(x), axis=1, keepdims=True)
        tgt = jnp.sum(jnp.where(cls_iota == lab, x, 0.0), axis=1, keepdims=True)
        ce = jnp.log(s) - tgt
        posm = lab != 0
        negacc = jnp.where(lane_nc == i, jnp.where(posm, 0.0, ce), negacc)
        return acc + jnp.sum(jnp.where(posm, ce, 0.0)), negacc

    acc, negacc = jax.lax.fori_loop(
        0, NCHUNK, body,
        (jnp.float32(0.0), jnp.zeros((CHUNK, NCHUNK), jnp.float32)))
    neg_ref[0] = negacc
    lane = jax.lax.broadcasted_iota(jnp.int32, (1, 128), 1)
    stats_ref[0] = jnp.where(lane == 0, acc, 0.0)


def _mine_kernel(neg_ref, s1_ref, s2_ref, out_ref):
    vals = neg_ref[...]
    bits = jax.lax.bitcast_convert_type(vals, jnp.int32)
    npos = s1_ref[:, :, 0:1]
    k = jnp.minimum(jnp.float32(NEG_POS) * npos, jnp.float32(N)).astype(jnp.int32)

    lo = jnp.zeros((B, 1, 1), jnp.int32)
    hi = jnp.full((B, 1, 1), 0x7F7FFFFF, jnp.int32)

    def body(i, c):
        lo, hi = c
        d = hi - lo
        mid = lo + (d >> 1) + (d & 1)
        cnt = jnp.sum((bits >= mid).astype(jnp.int32), axis=(1, 2), keepdims=True)
        good = cnt >= k
        lo = jnp.where(good, mid, lo)
        hi = jnp.where(good, hi, mid - 1)
        return lo, hi

    t, _ = jax.lax.fori_loop(0, 31, body, (lo, hi))
    above = bits > t
    m = jnp.sum(above.astype(jnp.float32), axis=(1, 2), keepdims=True)
    s = jnp.sum(jnp.where(above, vals, 0.0), axis=(1, 2), keepdims=True)
    tval = jax.lax.bitcast_convert_type(t, jnp.float32)
    hard = jnp.where(k > 0, s + (k.astype(jnp.float32) - m) * tval, 0.0)

    n_pos_tot = jnp.sum(npos)
    loc_num = jnp.sum(s1_ref[:, :, 1:2])
    conf_pos = jnp.sum(s2_ref[:, :, 0:1])
    hard_tot = jnp.sum(hard)
    loc_loss = loc_num / jnp.maximum(4.0 * n_pos_tot, 1.0)
    conf_loss = (hard_tot + conf_pos) / jnp.maximum(n_pos_tot, 1.0)
    lane = jax.lax.broadcasted_iota(jnp.int32, (1, 128), 1)
    out_ref[...] = jnp.where(lane == 0, loc_loss, jnp.where(lane == 1, conf_loss, 0.0))


def _sc_gather16(x, idx):
    return lax.gather(
        x, idx[:, None],
        lax.GatherDimensionNumbers(offset_dims=(), collapsed_slice_dims=(0,),
                                   start_index_map=(0,)),
        (1,), mode=lax.GatherScatterMode.PROMISE_IN_BOUNDS)


def _sc_mine(neg_hbm, stats_hbm, out_hbm, data_v, stats_v, pub_v, all_v, shared):
    c = lax.axis_index("c")
    s = lax.axis_index("s")
    lane = lax.iota(jnp.int32, 16)
    nchunk = N // 16

    def vtree(x, op):
        for kk in (8, 4, 2, 1):
            x = op(x, _sc_gather16(x, lane ^ kk))
        return x

    @pl.when(c == 0)
    def _work():
        pltpu.sync_copy(stats_hbm, stats_v)
        pltpu.sync_copy(neg_hbm.at[s], data_v)
        npos16 = stats_v[0]
        npos_w = vtree(jnp.where(lane == s, npos16, 0.0), jnp.add)
        k = jnp.minimum(jnp.float32(NEG_POS) * npos_w, jnp.float32(N))

        def cmax(i, acc):
            return jnp.maximum(acc, data_v[pl.ds(i * 16, 16)])

        vmax = vtree(lax.fori_loop(0, nchunk, cmax, jnp.zeros((16,), jnp.float32)),
                     jnp.maximum)

        def count_ge(mid):
            def chunk(i, acc):
                return acc + jnp.where(data_v[pl.ds(i * 16, 16)] >= mid, 1.0, 0.0)
            return vtree(lax.fori_loop(0, nchunk, chunk,
                                       jnp.zeros((16,), jnp.float32)), jnp.add)

        def step(_, carry):
            lo, hi = carry
            mid = 0.5 * (lo + hi)
            good = count_ge(mid) >= k
            return jnp.where(good, mid, lo), jnp.where(good, hi, mid)

        t, _ = lax.fori_loop(0, 38, step,
                             (jnp.zeros((16,), jnp.float32), vmax))

        def chunk2(i, carry):
            sa, ma = carry
            v = data_v[pl.ds(i * 16, 16)]
            gt = v > t
            return sa + jnp.where(gt, v, 0.0), ma + jnp.where(gt, 1.0, 0.0)

        sa, ma = lax.fori_loop(0, nchunk, chunk2,
                               (jnp.zeros((16,), jnp.float32),
                                jnp.zeros((16,), jnp.float32)))
        m = vtree(ma, jnp.add)
        stot = vtree(sa, jnp.add)
        hard = jnp.where(k > 0, stot + (k - m) * t, 0.0)
        pub_v[...] = jnp.where(lane == s, hard, 0.0)
        pltpu.sync_copy(pub_v, shared.at[s])

    plsc.subcore_barrier()

    @pl.when((c == 0) & (s == 0))
    def _agg():
        pltpu.sync_copy(shared, all_v)
        h16 = jnp.zeros((16,), jnp.float32)
        for i in range(16):
            h16 = h16 + all_v[i]
        npos_tot = vtree(stats_v[0], jnp.add)
        loc_num = vtree(stats_v[1], jnp.add)
        conf_pos = vtree(stats_v[2], jnp.add)
        hard_tot = vtree(h16, jnp.add)
        loc_loss = loc_num / jnp.maximum(4.0 * npos_tot, 1.0)
        conf_loss = (hard_tot + conf_pos) / jnp.maximum(npos_tot, 1.0)
        pub_v[...] = jnp.where(lane == 0, loc_loss,
                               jnp.where(lane == 1, conf_loss, 0.0))
        pltpu.sync_copy(pub_v, out_hbm)


@functools.cache
def _sc_mine_call():
    return functools.partial(
        pl.kernel,
        out_type=jax.ShapeDtypeStruct((16,), jnp.float32),
        mesh=plsc.VectorSubcoreMesh(core_axis_name="c", subcore_axis_name="s"),
        scratch_types=[
            pltpu.VMEM((N,), jnp.float32),
            pltpu.VMEM((4, 16), jnp.float32),
            pltpu.VMEM((16,), jnp.float32),
            pltpu.VMEM((16, 16), jnp.float32),
            pltpu.HBM((16, 16), jnp.float32),
        ],
    )(_sc_mine)


@jax.jit
def kernel(locs_pred, cls_pred, boxes, labels, default_boxes):
    db3 = default_boxes.T.reshape(4, AS, AL)
    boxesT = boxes.transpose(0, 2, 1)
    labels3 = labels.reshape(B, 1, NOBJ)
    locsT = locs_pred.transpose(0, 2, 1).reshape(B, 4, AS, AL)

    tcls, stats1 = pl.pallas_call(
        _match_kernel,
        grid=(B,),
        in_specs=[
            pl.BlockSpec((4, AS, AL), lambda i: (0, 0, 0)),
            pl.BlockSpec((1, 4, NOBJ), lambda i: (i, 0, 0), memory_space=pltpu.SMEM),
            pl.BlockSpec((1, 1, NOBJ), lambda i: (i, 0, 0), memory_space=pltpu.SMEM),
            pl.BlockSpec((1, 4, AS, AL), lambda i: (i, 0, 0, 0)),
        ],
        out_specs=[
            pl.BlockSpec((1, AS, AL), lambda i: (i, 0, 0)),
            pl.BlockSpec((1, 1, 128), lambda i: (i, 0, 0)),
        ],
        out_shape=[
            jax.ShapeDtypeStruct((B, AS, AL), jnp.int32),
            jax.ShapeDtypeStruct((B, 1, 128), jnp.float32),
        ],
        interpret=_INTERPRET,
    )(db3, boxesT, labels3, locsT)

    labT = tcls.reshape(B, NCHUNK, CHUNK).transpose(0, 2, 1)

    neg, stats2 = pl.pallas_call(
        _ce_kernel,
        grid=(B,),
        in_specs=[
            pl.BlockSpec((1, N, C), lambda i: (i, 0, 0)),
            pl.BlockSpec((1, CHUNK, NCHUNK), lambda i: (i, 0, 0)),
        ],
        out_specs=[
            pl.BlockSpec((1, CHUNK, NCHUNK), lambda i: (i, 0, 0)),
            pl.BlockSpec((1, 1, 128), lambda i: (i, 0, 0)),
        ],
        out_shape=[
            jax.ShapeDtypeStruct((B, CHUNK, NCHUNK), jnp.float32),
            jax.ShapeDtypeStruct((B, 1, 128), jnp.float32),
        ],
        interpret=_INTERPRET,
    )(cls_pred, labT)

    negd = neg.transpose(0, 2, 1).reshape(B, N)
    stats_sc = jnp.stack([
        stats1[:, 0, 0], stats1[:, 0, 1], stats2[:, 0, 0],
        jnp.zeros((B,), jnp.float32),
    ])

    out = _sc_mine_call()(negd, stats_sc)
    return (out[0], out[1])


# speedup vs baseline: 1.1969x; 1.1969x over previous
import functools

import jax
import jax.numpy as jnp
from jax import lax
from jax.experimental import pallas as pl
from jax.experimental.pallas import tpu as pltpu
from jax.experimental.pallas import tpu_sc as plsc

N = 20000
B = 16
NOBJ = 32
C = 81
THRESHOLD = 0.5
NEG_POS = 3
AS, AL = 160, 125
CHUNK, NCHUNK = 800, 25

_INTERPRET = False


def _match_kernel(db_ref, boxes_ref, labels_ref, locs_ref, tcls_ref, stats_ref):
    dcx = db_ref[0]
    dcy = db_ref[1]
    dw = db_ref[2]
    dh = db_ref[3]
    dx0 = dcx - dw / 2.0
    dy0 = dcy - dh / 2.0
    dx1 = dcx + dw / 2.0
    dy1 = dcy + dh / 2.0
    darea = (dx1 - dx0) * (dy1 - dy0)

    row_ids = jax.lax.broadcasted_iota(jnp.int32, (AS, AL), 0)
    col_ids = jax.lax.broadcasted_iota(jnp.int32, (AS, AL), 1)
    aidx = row_ids * AL + col_ids

    best = jnp.full((AS, AL), -1.0, jnp.float32)
    besti = jnp.zeros((AS, AL), jnp.int32)
    dbj = []
    for j in range(NOBJ):
        bx0 = boxes_ref[0, 0, j]
        by0 = boxes_ref[0, 1, j]
        bx1 = boxes_ref[0, 2, j]
        by1 = boxes_ref[0, 3, j]
        barea = (bx1 - bx0) * (by1 - by0)
        ix = jnp.maximum(jnp.minimum(bx1, dx1) - jnp.maximum(bx0, dx0), 0.0)
        iy = jnp.maximum(jnp.minimum(by1, dy1) - jnp.maximum(by0, dy0), 0.0)
        inter = ix * iy
        union = jnp.maximum(barea + darea - inter, 1e-10)
        iou = inter / union
        upd = iou > best
        besti = jnp.where(upd, j, besti)
        best = jnp.where(upd, iou, best)
        mx = jnp.max(iou)
        am = jnp.min(jnp.where(iou == mx, aidx, N))
        dbj.append(am)

    fj = jnp.full((AS, AL), -1, jnp.int32)
    for j in range(NOBJ):
        fj = jnp.where(aidx == dbj[j], j, fj)
    o = jnp.where(fj >= 0, fj, besti)
    ovl = jnp.where(fj >= 0, 1.0, best)

    lab = jnp.zeros((AS, AL), jnp.int32)
    gx0 = jnp.zeros((AS, AL), jnp.float32)
    gy0 = jnp.zeros((AS, AL), jnp.float32)
    gx1 = jnp.zeros((AS, AL), jnp.float32)
    gy1 = jnp.zeros((AS, AL), jnp.float32)
    for j in range(NOBJ):
        m = o == j
        lab = jnp.where(m, labels_ref[0, 0, j], lab)
        gx0 = jnp.where(m, boxes_ref[0, 0, j], gx0)
        gy0 = jnp.where(m, boxes_ref[0, 1, j], gy0)
        gx1 = jnp.where(m, boxes_ref[0, 2, j], gx1)
        gy1 = jnp.where(m, boxes_ref[0, 3, j], gy1)
    lab = jnp.where(ovl < THRESHOLD, 0, lab)
    tcls_ref[0] = lab

    pos = lab != 0
    n_pos = jnp.sum(pos.astype(jnp.float32))

    cxt = (gx0 + gx1) / 2.0
    cyt = (gy0 + gy1) / 2.0
    wt = gx1 - gx0
    ht = gy1 - gy0
    g0 = (cxt - dcx) / (dw / 10.0)
    g1 = (cyt - dcy) / (dh / 10.0)
    g2 = jnp.log(jnp.maximum(wt / dw, 1e-8)) * 5.0
    g3 = jnp.log(jnp.maximum(ht / dh, 1e-8)) * 5.0
    sl = jnp.zeros((AS, AL), jnp.float32)
    for c, g in enumerate((g0, g1, g2, g3)):
        ad = jnp.abs(locs_ref[0, c] - g)
        sl = sl + jnp.where(ad < 1.0, 0.5 * ad * ad, ad - 0.5)
    loc_num = jnp.sum(jnp.where(pos, sl, 0.0))

    lane = jax.lax.broadcasted_iota(jnp.int32, (1, 128), 1)
    stats_ref[0] = jnp.where(lane == 0, n_pos, jnp.where(lane == 1, loc_num, 0.0))


def _ce_kernel(cls_ref, lab_ref, neg_ref, stats_ref):
    cls_iota = jax.lax.broadcasted_iota(jnp.int32, (CHUNK, C), 1)
    lane_nc = jax.lax.broadcasted_iota(jnp.int32, (CHUNK, NCHUNK), 1)
    labfull = lab_ref[0]

    def body(i, carry):
        acc, negacc = carry
        x = cls_ref[0, pl.ds(i * CHUNK, CHUNK), :]
        lab = jnp.sum(jnp.where(lane_nc == i, labfull, 0), axis=1, keepdims=True)
        s = jnp.sum(jnp.exp(x), axis=1, keepdims=True)
        tgt = jnp.sum(jnp.where(cls_iota == lab, x, 0.0), axis=1, keepdims=True)
        ce = jnp.log(s) - tgt
        posm = lab != 0
        negacc = jnp.where(lane_nc == i, jnp.where(posm, 0.0, ce), negacc)
        return acc + jnp.sum(jnp.where(posm, ce, 0.0)), negacc

    acc, negacc = jax.lax.fori_loop(
        0, NCHUNK, body,
        (jnp.float32(0.0), jnp.zeros((CHUNK, NCHUNK), jnp.float32)))
    neg_ref[0] = negacc
    lane = jax.lax.broadcasted_iota(jnp.int32, (1, 128), 1)
    stats_ref[0] = jnp.where(lane == 0, acc, 0.0)


def _mine_kernel(neg_ref, s1_ref, s2_ref, out_ref):
    vals = neg_ref[...]
    bits = jax.lax.bitcast_convert_type(vals, jnp.int32)
    npos = s1_ref[:, :, 0:1]
    k = jnp.minimum(jnp.float32(NEG_POS) * npos, jnp.float32(N)).astype(jnp.int32)

    lo = jnp.zeros((B, 1, 1), jnp.int32)
    hi = jnp.full((B, 1, 1), 0x7F7FFFFF, jnp.int32)

    def body(i, c):
        lo, hi = c
        d = hi - lo
        mid = lo + (d >> 1) + (d & 1)
        cnt = jnp.sum((bits >= mid).astype(jnp.int32), axis=(1, 2), keepdims=True)
        good = cnt >= k
        lo = jnp.where(good, mid, lo)
        hi = jnp.where(good, hi, mid - 1)
        return lo, hi

    t, _ = jax.lax.fori_loop(0, 31, body, (lo, hi))
    above = bits > t
    m = jnp.sum(above.astype(jnp.float32), axis=(1, 2), keepdims=True)
    s = jnp.sum(jnp.where(above, vals, 0.0), axis=(1, 2), keepdims=True)
    tval = jax.lax.bitcast_convert_type(t, jnp.float32)
    hard = jnp.where(k > 0, s + (k.astype(jnp.float32) - m) * tval, 0.0)

    n_pos_tot = jnp.sum(npos)
    loc_num = jnp.sum(s1_ref[:, :, 1:2])
    conf_pos = jnp.sum(s2_ref[:, :, 0:1])
    hard_tot = jnp.sum(hard)
    loc_loss = loc_num / jnp.maximum(4.0 * n_pos_tot, 1.0)
    conf_loss = (hard_tot + conf_pos) / jnp.maximum(n_pos_tot, 1.0)
    lane = jax.lax.broadcasted_iota(jnp.int32, (1, 128), 1)
    out_ref[...] = jnp.where(lane == 0, loc_loss, jnp.where(lane == 1, conf_loss, 0.0))


def _sc_gather16(x, idx):
    return lax.gather(
        x, idx[:, None],
        lax.GatherDimensionNumbers(offset_dims=(), collapsed_slice_dims=(0,),
                                   start_index_map=(0,)),
        (1,), mode=lax.GatherScatterMode.PROMISE_IN_BOUNDS)


def _sc_mine(neg_hbm, stats_hbm, out_hbm, data_v, stats_v, pub_v, all_v, shared):
    c = lax.axis_index("c")
    s = lax.axis_index("s")
    lane = lax.iota(jnp.int32, 16)
    nchunk = N // 16

    def vtree(x, op):
        for kk in (8, 4, 2, 1):
            x = op(x, _sc_gather16(x, lane ^ kk))
        return x

    @pl.when(c == 0)
    def _work():
        pltpu.sync_copy(stats_hbm, stats_v)
        pltpu.sync_copy(neg_hbm.at[s], data_v)
        npos16 = stats_v[0]
        npos_w = vtree(jnp.where(lane == s, npos16, 0.0), jnp.add)
        k = jnp.minimum(jnp.float32(NEG_POS) * npos_w, jnp.float32(N))

        def cmax(i, acc):
            for u in range(4):
                acc = jnp.maximum(acc, data_v[pl.ds(i * 64 + u * 16, 16)])
            return acc

        vmax = vtree(lax.fori_loop(0, nchunk // 4, cmax,
                                   jnp.zeros((16,), jnp.float32)), jnp.maximum)

        def count_ge(mid):
            def chunk(i, acc):
                for u in range(4):
                    acc = acc + jnp.where(
                        data_v[pl.ds(i * 64 + u * 16, 16)] >= mid, 1.0, 0.0)
                return acc
            return vtree(lax.fori_loop(0, nchunk // 4, chunk,
                                       jnp.zeros((16,), jnp.float32)), jnp.add)

        def step(_, carry):
            lo, hi = carry
            mid = 0.5 * (lo + hi)
            good = count_ge(mid) >= k
            return jnp.where(good, mid, lo), jnp.where(good, hi, mid)

        t, _ = lax.fori_loop(0, 31, step,
                             (jnp.zeros((16,), jnp.float32), vmax))

        def chunk2(i, carry):
            sa, ma = carry
            for u in range(4):
                v = data_v[pl.ds(i * 64 + u * 16, 16)]
                gt = v > t
                sa = sa + jnp.where(gt, v, 0.0)
                ma = ma + jnp.where(gt, 1.0, 0.0)
            return sa, ma

        sa, ma = lax.fori_loop(0, nchunk // 4, chunk2,
                               (jnp.zeros((16,), jnp.float32),
                                jnp.zeros((16,), jnp.float32)))
        m = vtree(ma, jnp.add)
        stot = vtree(sa, jnp.add)
        hard = jnp.where(k > 0, stot + (k - m) * t, 0.0)
        pub_v[...] = jnp.where(lane == s, hard, 0.0)
        pltpu.sync_copy(pub_v, shared.at[s])

    plsc.subcore_barrier()

    @pl.when((c == 0) & (s == 0))
    def _agg():
        pltpu.sync_copy(shared, all_v)
        h16 = jnp.zeros((16,), jnp.float32)
        for i in range(16):
            h16 = h16 + all_v[i]
        npos_tot = vtree(stats_v[0], jnp.add)
        loc_num = vtree(stats_v[1], jnp.add)
        conf_pos = vtree(stats_v[2], jnp.add)
        hard_tot = vtree(h16, jnp.add)
        loc_loss = loc_num / jnp.maximum(4.0 * npos_tot, 1.0)
        conf_loss = (hard_tot + conf_pos) / jnp.maximum(npos_tot, 1.0)
        pub_v[...] = jnp.where(lane == 0, loc_loss,
                               jnp.where(lane == 1, conf_loss, 0.0))
        pltpu.sync_copy(pub_v, out_hbm)


@functools.cache
def _sc_mine_call():
    return functools.partial(
        pl.kernel,
        out_type=jax.ShapeDtypeStruct((16,), jnp.float32),
        mesh=plsc.VectorSubcoreMesh(core_axis_name="c", subcore_axis_name="s"),
        scratch_types=[
            pltpu.VMEM((N,), jnp.float32),
            pltpu.VMEM((4, 16), jnp.float32),
            pltpu.VMEM((16,), jnp.float32),
            pltpu.VMEM((16, 16), jnp.float32),
            pltpu.HBM((16, 16), jnp.float32),
        ],
    )(_sc_mine)


@jax.jit
def kernel(locs_pred, cls_pred, boxes, labels, default_boxes):
    db3 = default_boxes.T.reshape(4, AS, AL)
    boxesT = boxes.transpose(0, 2, 1)
    labels3 = labels.reshape(B, 1, NOBJ)
    locsT = locs_pred.transpose(0, 2, 1).reshape(B, 4, AS, AL)

    tcls, stats1 = pl.pallas_call(
        _match_kernel,
        grid=(B,),
        in_specs=[
            pl.BlockSpec((4, AS, AL), lambda i: (0, 0, 0)),
            pl.BlockSpec((1, 4, NOBJ), lambda i: (i, 0, 0), memory_space=pltpu.SMEM),
            pl.BlockSpec((1, 1, NOBJ), lambda i: (i, 0, 0), memory_space=pltpu.SMEM),
            pl.BlockSpec((1, 4, AS, AL), lambda i: (i, 0, 0, 0)),
        ],
        out_specs=[
            pl.BlockSpec((1, AS, AL), lambda i: (i, 0, 0)),
            pl.BlockSpec((1, 1, 128), lambda i: (i, 0, 0)),
        ],
        out_shape=[
            jax.ShapeDtypeStruct((B, AS, AL), jnp.int32),
            jax.ShapeDtypeStruct((B, 1, 128), jnp.float32),
        ],
        interpret=_INTERPRET,
    )(db3, boxesT, labels3, locsT)

    labT = tcls.reshape(B, NCHUNK, CHUNK).transpose(0, 2, 1)

    neg, stats2 = pl.pallas_call(
        _ce_kernel,
        grid=(B,),
        in_specs=[
            pl.BlockSpec((1, N, C), lambda i: (i, 0, 0)),
            pl.BlockSpec((1, CHUNK, NCHUNK), lambda i: (i, 0, 0)),
        ],
        out_specs=[
            pl.BlockSpec((1, CHUNK, NCHUNK), lambda i: (i, 0, 0)),
            pl.BlockSpec((1, 1, 128), lambda i: (i, 0, 0)),
        ],
        out_shape=[
            jax.ShapeDtypeStruct((B, CHUNK, NCHUNK), jnp.float32),
            jax.ShapeDtypeStruct((B, 1, 128), jnp.float32),
        ],
        interpret=_INTERPRET,
    )(cls_pred, labT)

    negd = neg.transpose(0, 2, 1).reshape(B, N)
    stats_sc = jnp.stack([
        stats1[:, 0, 0], stats1[:, 0, 1], stats2[:, 0, 0],
        jnp.zeros((B,), jnp.float32),
    ])

    out = _sc_mine_call()(negd, stats_sc)
    return (out[0], out[1])


# speedup vs baseline: 1.3416x; 1.1210x over previous
import functools

import jax
import jax.numpy as jnp
from jax import lax
from jax.experimental import pallas as pl
from jax.experimental.pallas import tpu as pltpu
from jax.experimental.pallas import tpu_sc as plsc

N = 20000
B = 16
NOBJ = 32
C = 81
THRESHOLD = 0.5
NEG_POS = 3
AS, AL = 160, 125
CHUNK, NCHUNK = 800, 25

_INTERPRET = False


def _match_kernel(db_ref, boxes_ref, labels_ref, locs_ref, tcls_ref, stats_ref,
                  iou_ref):
    dcx = db_ref[0]
    dcy = db_ref[1]
    dw = db_ref[2]
    dh = db_ref[3]
    dx0 = dcx - dw / 2.0
    dy0 = dcy - dh / 2.0
    dx1 = dcx + dw / 2.0
    dy1 = dcy + dh / 2.0
    darea = (dx1 - dx0) * (dy1 - dy0)

    row_ids = jax.lax.broadcasted_iota(jnp.int32, (AS, AL), 0)
    col_ids = jax.lax.broadcasted_iota(jnp.int32, (AS, AL), 1)
    aidx = row_ids * AL + col_ids

    best = jnp.full((AS, AL), -1.0, jnp.float32)
    besti = jnp.zeros((AS, AL), jnp.int32)
    for j in range(NOBJ):
        bx0 = boxes_ref[0, 0, j]
        by0 = boxes_ref[0, 1, j]
        bx1 = boxes_ref[0, 2, j]
        by1 = boxes_ref[0, 3, j]
        barea = (bx1 - bx0) * (by1 - by0)
        ix = jnp.maximum(jnp.minimum(bx1, dx1) - jnp.maximum(bx0, dx0), 0.0)
        iy = jnp.maximum(jnp.minimum(by1, dy1) - jnp.maximum(by0, dy0), 0.0)
        inter = ix * iy
        union = jnp.maximum(barea + darea - inter, 1e-10)
        iou = inter / union
        upd = iou > best
        besti = jnp.where(upd, j, besti)
        best = jnp.where(upd, iou, best)
        iou_ref[j] = iou

    iou_all = iou_ref[...]
    rowmax = jnp.max(iou_all, axis=(1, 2), keepdims=True)
    cand = jnp.where(iou_all == rowmax, aidx[None], N)
    dbj3 = jnp.min(cand, axis=(1, 2), keepdims=True)
    jidx = jax.lax.broadcasted_iota(jnp.int32, (NOBJ, 1, 1), 0)
    fcand = jnp.where(aidx[None] == dbj3, jidx, -1)
    fj = jnp.max(fcand, axis=0)
    o = jnp.where(fj >= 0, fj, besti)
    ovl = jnp.where(fj >= 0, 1.0, best)

    lab = jnp.zeros((AS, AL), jnp.int32)
    gx0 = jnp.zeros((AS, AL), jnp.float32)
    gy0 = jnp.zeros((AS, AL), jnp.float32)
    gx1 = jnp.zeros((AS, AL), jnp.float32)
    gy1 = jnp.zeros((AS, AL), jnp.float32)
    for j in range(NOBJ):
        m = o == j
        lab = jnp.where(m, labels_ref[0, 0, j], lab)
        gx0 = jnp.where(m, boxes_ref[0, 0, j], gx0)
        gy0 = jnp.where(m, boxes_ref[0, 1, j], gy0)
        gx1 = jnp.where(m, boxes_ref[0, 2, j], gx1)
        gy1 = jnp.where(m, boxes_ref[0, 3, j], gy1)
    lab = jnp.where(ovl < THRESHOLD, 0, lab)
    tcls_ref[0] = lab

    pos = lab != 0
    n_pos = jnp.sum(pos.astype(jnp.float32))

    cxt = (gx0 + gx1) / 2.0
    cyt = (gy0 + gy1) / 2.0
    wt = gx1 - gx0
    ht = gy1 - gy0
    g0 = (cxt - dcx) / (dw / 10.0)
    g1 = (cyt - dcy) / (dh / 10.0)
    g2 = jnp.log(jnp.maximum(wt / dw, 1e-8)) * 5.0
    g3 = jnp.log(jnp.maximum(ht / dh, 1e-8)) * 5.0
    sl = jnp.zeros((AS, AL), jnp.float32)
    for c, g in enumerate((g0, g1, g2, g3)):
        ad = jnp.abs(locs_ref[0, c] - g)
        sl = sl + jnp.where(ad < 1.0, 0.5 * ad * ad, ad - 0.5)
    loc_num = jnp.sum(jnp.where(pos, sl, 0.0))

    lane = jax.lax.broadcasted_iota(jnp.int32, (1, 128), 1)
    stats_ref[0] = jnp.where(lane == 0, n_pos, jnp.where(lane == 1, loc_num, 0.0))


def _ce_kernel(cls_ref, lab_ref, neg_ref, stats_ref):
    cls_iota = jax.lax.broadcasted_iota(jnp.int32, (CHUNK, C), 1)
    lane_nc = jax.lax.broadcasted_iota(jnp.int32, (CHUNK, NCHUNK), 1)
    labfull = lab_ref[0]

    def body(i, carry):
        acc, negacc = carry
        x = cls_ref[0, pl.ds(i * CHUNK, CHUNK), :]
        lab = jnp.sum(jnp.where(lane_nc == i, labfull, 0), axis=1, keepdims=True)
        s = jnp.sum(jnp.exp(x), axis=1, keepdims=True)
        tgt = jnp.sum(jnp.where(cls_iota == lab, x, 0.0), axis=1, keepdims=True)
        ce = jnp.log(s) - tgt
        posm = lab != 0
        negacc = jnp.where(lane_nc == i, jnp.where(posm, 0.0, ce), negacc)
        return acc + jnp.sum(jnp.where(posm, ce, 0.0)), negacc

    acc, negacc = jax.lax.fori_loop(
        0, NCHUNK, body,
        (jnp.float32(0.0), jnp.zeros((CHUNK, NCHUNK), jnp.float32)))
    neg_ref[0] = negacc
    lane = jax.lax.broadcasted_iota(jnp.int32, (1, 128), 1)
    stats_ref[0] = jnp.where(lane == 0, acc, 0.0)


def _mine_kernel(neg_ref, s1_ref, s2_ref, out_ref):
    vals = neg_ref[...]
    bits = jax.lax.bitcast_convert_type(vals, jnp.int32)
    npos = s1_ref[:, :, 0:1]
    k = jnp.minimum(jnp.float32(NEG_POS) * npos, jnp.float32(N)).astype(jnp.int32)

    lo = jnp.zeros((B, 1, 1), jnp.int32)
    hi = jnp.full((B, 1, 1), 0x7F7FFFFF, jnp.int32)

    def body(i, c):
        lo, hi = c
        d = hi - lo
        mid = lo + (d >> 1) + (d & 1)
        cnt = jnp.sum((bits >= mid).astype(jnp.int32), axis=(1, 2), keepdims=True)
        good = cnt >= k
        lo = jnp.where(good, mid, lo)
        hi = jnp.where(good, hi, mid - 1)
        return lo, hi

    t, _ = jax.lax.fori_loop(0, 31, body, (lo, hi))
    above = bits > t
    m = jnp.sum(above.astype(jnp.float32), axis=(1, 2), keepdims=True)
    s = jnp.sum(jnp.where(above, vals, 0.0), axis=(1, 2), keepdims=True)
    tval = jax.lax.bitcast_convert_type(t, jnp.float32)
    hard = jnp.where(k > 0, s + (k.astype(jnp.float32) - m) * tval, 0.0)

    n_pos_tot = jnp.sum(npos)
    loc_num = jnp.sum(s1_ref[:, :, 1:2])
    conf_pos = jnp.sum(s2_ref[:, :, 0:1])
    hard_tot = jnp.sum(hard)
    loc_loss = loc_num / jnp.maximum(4.0 * n_pos_tot, 1.0)
    conf_loss = (hard_tot + conf_pos) / jnp.maximum(n_pos_tot, 1.0)
    lane = jax.lax.broadcasted_iota(jnp.int32, (1, 128), 1)
    out_ref[...] = jnp.where(lane == 0, loc_loss, jnp.where(lane == 1, conf_loss, 0.0))


def _sc_gather16(x, idx):
    return lax.gather(
        x, idx[:, None],
        lax.GatherDimensionNumbers(offset_dims=(), collapsed_slice_dims=(0,),
                                   start_index_map=(0,)),
        (1,), mode=lax.GatherScatterMode.PROMISE_IN_BOUNDS)


def _sc_mine(neg_hbm, stats_hbm, out_hbm, data_v, stats_v, pub_v, all_v, shared):
    c = lax.axis_index("c")
    s = lax.axis_index("s")
    lane = lax.iota(jnp.int32, 16)
    nchunk = N // 16

    def vtree(x, op):
        for kk in (8, 4, 2, 1):
            x = op(x, _sc_gather16(x, lane ^ kk))
        return x

    @pl.when(c == 0)
    def _work():
        pltpu.sync_copy(stats_hbm, stats_v)
        pltpu.sync_copy(neg_hbm.at[s], data_v)
        npos16 = stats_v[0]
        npos_w = vtree(jnp.where(lane == s, npos16, 0.0), jnp.add)
        k = jnp.minimum(jnp.float32(NEG_POS) * npos_w, jnp.float32(N))

        def cmax(i, acc):
            for u in range(4):
                acc = jnp.maximum(acc, data_v[pl.ds(i * 64 + u * 16, 16)])
            return acc

        vmax = vtree(lax.fori_loop(0, nchunk // 4, cmax,
                                   jnp.zeros((16,), jnp.float32)), jnp.maximum)

        def count_ge(mid):
            def chunk(i, acc):
                for u in range(4):
                    acc = acc + jnp.where(
                        data_v[pl.ds(i * 64 + u * 16, 16)] >= mid, 1.0, 0.0)
                return acc
            return vtree(lax.fori_loop(0, nchunk // 4, chunk,
                                       jnp.zeros((16,), jnp.float32)), jnp.add)

        def step(_, carry):
            lo, hi = carry
            mid = 0.5 * (lo + hi)
            good = count_ge(mid) >= k
            return jnp.where(good, mid, lo), jnp.where(good, hi, mid)

        t, _ = lax.fori_loop(0, 31, step,
                             (jnp.zeros((16,), jnp.float32), vmax))

        def chunk2(i, carry):
            sa, ma = carry
            for u in range(4):
                v = data_v[pl.ds(i * 64 + u * 16, 16)]
                gt = v > t
                sa = sa + jnp.where(gt, v, 0.0)
                ma = ma + jnp.where(gt, 1.0, 0.0)
            return sa, ma

        sa, ma = lax.fori_loop(0, nchunk // 4, chunk2,
                               (jnp.zeros((16,), jnp.float32),
                                jnp.zeros((16,), jnp.float32)))
        m = vtree(ma, jnp.add)
        stot = vtree(sa, jnp.add)
        hard = jnp.where(k > 0, stot + (k - m) * t, 0.0)
        pub_v[...] = jnp.where(lane == s, hard, 0.0)
        pltpu.sync_copy(pub_v, shared.at[s])

    plsc.subcore_barrier()

    @pl.when((c == 0) & (s == 0))
    def _agg():
        pltpu.sync_copy(shared, all_v)
        h16 = jnp.zeros((16,), jnp.float32)
        for i in range(16):
            h16 = h16 + all_v[i]
        npos_tot = vtree(stats_v[0], jnp.add)
        loc_num = vtree(stats_v[1], jnp.add)
        conf_pos = vtree(stats_v[2], jnp.add)
        hard_tot = vtree(h16, jnp.add)
        loc_loss = loc_num / jnp.maximum(4.0 * npos_tot, 1.0)
        conf_loss = (hard_tot + conf_pos) / jnp.maximum(npos_tot, 1.0)
        pub_v[...] = jnp.where(lane == 0, loc_loss,
                               jnp.where(lane == 1, conf_loss, 0.0))
        pltpu.sync_copy(pub_v, out_hbm)


@functools.cache
def _sc_mine_call():
    return functools.partial(
        pl.kernel,
        out_type=jax.ShapeDtypeStruct((16,), jnp.float32),
        mesh=plsc.VectorSubcoreMesh(core_axis_name="c", subcore_axis_name="s"),
        scratch_types=[
            pltpu.VMEM((N,), jnp.float32),
            pltpu.VMEM((4, 16), jnp.float32),
            pltpu.VMEM((16,), jnp.float32),
            pltpu.VMEM((16, 16), jnp.float32),
            pltpu.HBM((16, 16), jnp.float32),
        ],
    )(_sc_mine)


@jax.jit
def kernel(locs_pred, cls_pred, boxes, labels, default_boxes):
    db3 = default_boxes.T.reshape(4, AS, AL)
    boxesT = boxes.transpose(0, 2, 1)
    labels3 = labels.reshape(B, 1, NOBJ)
    locsT = locs_pred.transpose(0, 2, 1).reshape(B, 4, AS, AL)

    tcls, stats1 = pl.pallas_call(
        _match_kernel,
        grid=(B,),
        in_specs=[
            pl.BlockSpec((4, AS, AL), lambda i: (0, 0, 0)),
            pl.BlockSpec((1, 4, NOBJ), lambda i: (i, 0, 0), memory_space=pltpu.SMEM),
            pl.BlockSpec((1, 1, NOBJ), lambda i: (i, 0, 0), memory_space=pltpu.SMEM),
            pl.BlockSpec((1, 4, AS, AL), lambda i: (i, 0, 0, 0)),
        ],
        out_specs=[
            pl.BlockSpec((1, AS, AL), lambda i: (i, 0, 0)),
            pl.BlockSpec((1, 1, 128), lambda i: (i, 0, 0)),
        ],
        out_shape=[
            jax.ShapeDtypeStruct((B, AS, AL), jnp.int32),
            jax.ShapeDtypeStruct((B, 1, 128), jnp.float32),
        ],
        scratch_shapes=[pltpu.VMEM((NOBJ, AS, AL), jnp.float32)],
        interpret=_INTERPRET,
    )(db3, boxesT, labels3, locsT)

    labT = tcls.reshape(B, NCHUNK, CHUNK).transpose(0, 2, 1)

    neg, stats2 = pl.pallas_call(
        _ce_kernel,
        grid=(B,),
        in_specs=[
            pl.BlockSpec((1, N, C), lambda i: (i, 0, 0)),
            pl.BlockSpec((1, CHUNK, NCHUNK), lambda i: (i, 0, 0)),
        ],
        out_specs=[
            pl.BlockSpec((1, CHUNK, NCHUNK), lambda i: (i, 0, 0)),
            pl.BlockSpec((1, 1, 128), lambda i: (i, 0, 0)),
        ],
        out_shape=[
            jax.ShapeDtypeStruct((B, CHUNK, NCHUNK), jnp.float32),
            jax.ShapeDtypeStruct((B, 1, 128), jnp.float32),
        ],
        interpret=_INTERPRET,
    )(cls_pred, labT)

    negd = neg.transpose(0, 2, 1).reshape(B, N)
    stats_sc = jnp.stack([
        stats1[:, 0, 0], stats1[:, 0, 1], stats2[:, 0, 0],
        jnp.zeros((B,), jnp.float32),
    ])

    out = _sc_mine_call()(negd, stats_sc)
    return (out[0], out[1])


# speedup vs baseline: 1.3510x; 1.0070x over previous
import functools

import jax
import jax.numpy as jnp
from jax import lax
from jax.experimental import pallas as pl
from jax.experimental.pallas import tpu as pltpu
from jax.experimental.pallas import tpu_sc as plsc

N = 20000
B = 16
NOBJ = 32
C = 81
THRESHOLD = 0.5
NEG_POS = 3
AS, AL = 160, 125
CHUNK, NCHUNK = 800, 25

_INTERPRET = False


def _match_kernel(db_ref, boxes_ref, labels_ref, locs_ref, tcls_ref, stats_ref,
                  iou_ref):
    dcx = db_ref[0]
    dcy = db_ref[1]
    dw = db_ref[2]
    dh = db_ref[3]
    dx0 = dcx - dw / 2.0
    dy0 = dcy - dh / 2.0
    dx1 = dcx + dw / 2.0
    dy1 = dcy + dh / 2.0
    darea = (dx1 - dx0) * (dy1 - dy0)

    row_ids = jax.lax.broadcasted_iota(jnp.int32, (AS, AL), 0)
    col_ids = jax.lax.broadcasted_iota(jnp.int32, (AS, AL), 1)
    aidx = row_ids * AL + col_ids

    best = jnp.full((AS, AL), -1.0, jnp.float32)
    besti = jnp.zeros((AS, AL), jnp.int32)
    for j in range(NOBJ):
        bx0 = boxes_ref[0, 0, j]
        by0 = boxes_ref[0, 1, j]
        bx1 = boxes_ref[0, 2, j]
        by1 = boxes_ref[0, 3, j]
        barea = (bx1 - bx0) * (by1 - by0)
        ix = jnp.maximum(jnp.minimum(bx1, dx1) - jnp.maximum(bx0, dx0), 0.0)
        iy = jnp.maximum(jnp.minimum(by1, dy1) - jnp.maximum(by0, dy0), 0.0)
        inter = ix * iy
        union = jnp.maximum(barea + darea - inter, 1e-10)
        iou = inter / union
        upd = iou > best
        besti = jnp.where(upd, j, besti)
        best = jnp.where(upd, iou, best)
        iou_ref[j] = iou

    iou_all = iou_ref[...]
    rowmax = jnp.max(iou_all, axis=(1, 2), keepdims=True)
    cand = jnp.where(iou_all == rowmax, aidx[None], N)
    dbj3 = jnp.min(cand, axis=(1, 2), keepdims=True)
    jidx = jax.lax.broadcasted_iota(jnp.int32, (NOBJ, 1, 1), 0)
    fcand = jnp.where(aidx[None] == dbj3, jidx, -1)
    fj = jnp.max(fcand, axis=0)
    o = jnp.where(fj >= 0, fj, besti)
    ovl = jnp.where(fj >= 0, 1.0, best)

    lab = jnp.zeros((AS, AL), jnp.int32)
    gx0 = jnp.zeros((AS, AL), jnp.float32)
    gy0 = jnp.zeros((AS, AL), jnp.float32)
    gx1 = jnp.zeros((AS, AL), jnp.float32)
    gy1 = jnp.zeros((AS, AL), jnp.float32)
    for j in range(NOBJ):
        m = o == j
        lab = jnp.where(m, labels_ref[0, 0, j], lab)
        gx0 = jnp.where(m, boxes_ref[0, 0, j], gx0)
        gy0 = jnp.where(m, boxes_ref[0, 1, j], gy0)
        gx1 = jnp.where(m, boxes_ref[0, 2, j], gx1)
        gy1 = jnp.where(m, boxes_ref[0, 3, j], gy1)
    lab = jnp.where(ovl < THRESHOLD, 0, lab)
    tcls_ref[0] = lab

    pos = lab != 0
    n_pos = jnp.sum(pos.astype(jnp.float32))

    cxt = (gx0 + gx1) / 2.0
    cyt = (gy0 + gy1) / 2.0
    wt = gx1 - gx0
    ht = gy1 - gy0
    g0 = (cxt - dcx) / (dw / 10.0)
    g1 = (cyt - dcy) / (dh / 10.0)
    g2 = jnp.log(jnp.maximum(wt / dw, 1e-8)) * 5.0
    g3 = jnp.log(jnp.maximum(ht / dh, 1e-8)) * 5.0
    sl = jnp.zeros((AS, AL), jnp.float32)
    for c, g in enumerate((g0, g1, g2, g3)):
        ad = jnp.abs(locs_ref[0, c] - g)
        sl = sl + jnp.where(ad < 1.0, 0.5 * ad * ad, ad - 0.5)
    loc_num = jnp.sum(jnp.where(pos, sl, 0.0))

    lane = jax.lax.broadcasted_iota(jnp.int32, (1, 128), 1)
    stats_ref[0] = jnp.where(lane == 0, n_pos, jnp.where(lane == 1, loc_num, 0.0))


def _ce_kernel(cls_ref, lab_ref, neg_ref, stats_ref):
    cls_iota = jax.lax.broadcasted_iota(jnp.int32, (CHUNK, C), 1)
    acc = jnp.float32(0.0)
    for i in range(NCHUNK):
        x = cls_ref[0, pl.ds(i * CHUNK, CHUNK), :]
        lab = lab_ref[0, :, i:i + 1]
        s = jnp.sum(jnp.exp(x), axis=1, keepdims=True)
        tgt = jnp.sum(jnp.where(cls_iota == lab, x, 0.0), axis=1, keepdims=True)
        ce = jnp.log(s) - tgt
        posm = lab != 0
        neg_ref[0, :, i:i + 1] = jnp.where(posm, 0.0, ce)
        acc = acc + jnp.sum(jnp.where(posm, ce, 0.0))
    lane = jax.lax.broadcasted_iota(jnp.int32, (1, 128), 1)
    stats_ref[0] = jnp.where(lane == 0, acc, 0.0)


def _mine_kernel(neg_ref, s1_ref, s2_ref, out_ref):
    vals = neg_ref[...]
    bits = jax.lax.bitcast_convert_type(vals, jnp.int32)
    npos = s1_ref[:, :, 0:1]
    k = jnp.minimum(jnp.float32(NEG_POS) * npos, jnp.float32(N)).astype(jnp.int32)

    lo = jnp.zeros((B, 1, 1), jnp.int32)
    hi = jnp.full((B, 1, 1), 0x7F7FFFFF, jnp.int32)

    def body(i, c):
        lo, hi = c
        d = hi - lo
        mid = lo + (d >> 1) + (d & 1)
        cnt = jnp.sum((bits >= mid).astype(jnp.int32), axis=(1, 2), keepdims=True)
        good = cnt >= k
        lo = jnp.where(good, mid, lo)
        hi = jnp.where(good, hi, mid - 1)
        return lo, hi

    t, _ = jax.lax.fori_loop(0, 31, body, (lo, hi))
    above = bits > t
    m = jnp.sum(above.astype(jnp.float32), axis=(1, 2), keepdims=True)
    s = jnp.sum(jnp.where(above, vals, 0.0), axis=(1, 2), keepdims=True)
    tval = jax.lax.bitcast_convert_type(t, jnp.float32)
    hard = jnp.where(k > 0, s + (k.astype(jnp.float32) - m) * tval, 0.0)

    n_pos_tot = jnp.sum(npos)
    loc_num = jnp.sum(s1_ref[:, :, 1:2])
    conf_pos = jnp.sum(s2_ref[:, :, 0:1])
    hard_tot = jnp.sum(hard)
    loc_loss = loc_num / jnp.maximum(4.0 * n_pos_tot, 1.0)
    conf_loss = (hard_tot + conf_pos) / jnp.maximum(n_pos_tot, 1.0)
    lane = jax.lax.broadcasted_iota(jnp.int32, (1, 128), 1)
    out_ref[...] = jnp.where(lane == 0, loc_loss, jnp.where(lane == 1, conf_loss, 0.0))


def _sc_gather16(x, idx):
    return lax.gather(
        x, idx[:, None],
        lax.GatherDimensionNumbers(offset_dims=(), collapsed_slice_dims=(0,),
                                   start_index_map=(0,)),
        (1,), mode=lax.GatherScatterMode.PROMISE_IN_BOUNDS)


def _sc_mine(neg_hbm, stats_hbm, out_hbm, data_v, stats_v, pub_v, all_v, shared):
    c = lax.axis_index("c")
    s = lax.axis_index("s")
    lane = lax.iota(jnp.int32, 16)
    nchunk = N // 16

    def vtree(x, op):
        for kk in (8, 4, 2, 1):
            x = op(x, _sc_gather16(x, lane ^ kk))
        return x

    @pl.when(c == 0)
    def _work():
        pltpu.sync_copy(stats_hbm, stats_v)
        pltpu.sync_copy(neg_hbm.at[s], data_v)
        npos16 = stats_v[0]
        npos_w = vtree(jnp.where(lane == s, npos16, 0.0), jnp.add)
        k = jnp.minimum(jnp.float32(NEG_POS) * npos_w, jnp.float32(N))

        def cmax(i, acc):
            for u in range(4):
                acc = jnp.maximum(acc, data_v[pl.ds(i * 64 + u * 16, 16)])
            return acc

        vmax = vtree(lax.fori_loop(0, nchunk // 4, cmax,
                                   jnp.zeros((16,), jnp.float32)), jnp.maximum)

        def count_ge(mid):
            def chunk(i, acc):
                for u in range(4):
                    acc = acc + jnp.where(
                        data_v[pl.ds(i * 64 + u * 16, 16)] >= mid, 1.0, 0.0)
                return acc
            return vtree(lax.fori_loop(0, nchunk // 4, chunk,
                                       jnp.zeros((16,), jnp.float32)), jnp.add)

        def step(_, carry):
            lo, hi = carry
            mid = 0.5 * (lo + hi)
            good = count_ge(mid) >= k
            return jnp.where(good, mid, lo), jnp.where(good, hi, mid)

        t, _ = lax.fori_loop(0, 31, step,
                             (jnp.zeros((16,), jnp.float32), vmax))

        def chunk2(i, carry):
            sa, ma = carry
            for u in range(4):
                v = data_v[pl.ds(i * 64 + u * 16, 16)]
                gt = v > t
                sa = sa + jnp.where(gt, v, 0.0)
                ma = ma + jnp.where(gt, 1.0, 0.0)
            return sa, ma

        sa, ma = lax.fori_loop(0, nchunk // 4, chunk2,
                               (jnp.zeros((16,), jnp.float32),
                                jnp.zeros((16,), jnp.float32)))
        m = vtree(ma, jnp.add)
        stot = vtree(sa, jnp.add)
        hard = jnp.where(k > 0, stot + (k - m) * t, 0.0)
        pub_v[...] = jnp.where(lane == s, hard, 0.0)
        pltpu.sync_copy(pub_v, shared.at[s])

    plsc.subcore_barrier()

    @pl.when((c == 0) & (s == 0))
    def _agg():
        pltpu.sync_copy(shared, all_v)
        h16 = jnp.zeros((16,), jnp.float32)
        for i in range(16):
            h16 = h16 + all_v[i]
        npos_tot = vtree(stats_v[0], jnp.add)
        loc_num = vtree(stats_v[1], jnp.add)
        conf_pos = vtree(stats_v[2], jnp.add)
        hard_tot = vtree(h16, jnp.add)
        loc_loss = loc_num / jnp.maximum(4.0 * npos_tot, 1.0)
        conf_loss = (hard_tot + conf_pos) / jnp.maximum(npos_tot, 1.0)
        pub_v[...] = jnp.where(lane == 0, loc_loss,
                               jnp.where(lane == 1, conf_loss, 0.0))
        pltpu.sync_copy(pub_v, out_hbm)


@functools.cache
def _sc_mine_call():
    return functools.partial(
        pl.kernel,
        out_type=jax.ShapeDtypeStruct((16,), jnp.float32),
        mesh=plsc.VectorSubcoreMesh(core_axis_name="c", subcore_axis_name="s"),
        scratch_types=[
            pltpu.VMEM((N,), jnp.float32),
            pltpu.VMEM((4, 16), jnp.float32),
            pltpu.VMEM((16,), jnp.float32),
            pltpu.VMEM((16, 16), jnp.float32),
            pltpu.HBM((16, 16), jnp.float32),
        ],
    )(_sc_mine)


@jax.jit
def kernel(locs_pred, cls_pred, boxes, labels, default_boxes):
    db3 = default_boxes.T.reshape(4, AS, AL)
    boxesT = boxes.transpose(0, 2, 1)
    labels3 = labels.reshape(B, 1, NOBJ)
    locsT = locs_pred.transpose(0, 2, 1).reshape(B, 4, AS, AL)

    tcls, stats1 = pl.pallas_call(
        _match_kernel,
        grid=(B,),
        in_specs=[
            pl.BlockSpec((4, AS, AL), lambda i: (0, 0, 0)),
            pl.BlockSpec((1, 4, NOBJ), lambda i: (i, 0, 0), memory_space=pltpu.SMEM),
            pl.BlockSpec((1, 1, NOBJ), lambda i: (i, 0, 0), memory_space=pltpu.SMEM),
            pl.BlockSpec((1, 4, AS, AL), lambda i: (i, 0, 0, 0)),
        ],
        out_specs=[
            pl.BlockSpec((1, AS, AL), lambda i: (i, 0, 0)),
            pl.BlockSpec((1, 1, 128), lambda i: (i, 0, 0)),
        ],
        out_shape=[
            jax.ShapeDtypeStruct((B, AS, AL), jnp.int32),
            jax.ShapeDtypeStruct((B, 1, 128), jnp.float32),
        ],
        scratch_shapes=[pltpu.VMEM((NOBJ, AS, AL), jnp.float32)],
        interpret=_INTERPRET,
    )(db3, boxesT, labels3, locsT)

    labT = tcls.reshape(B, NCHUNK, CHUNK).transpose(0, 2, 1)

    neg, stats2 = pl.pallas_call(
        _ce_kernel,
        grid=(B,),
        in_specs=[
            pl.BlockSpec((1, N, C), lambda i: (i, 0, 0)),
            pl.BlockSpec((1, CHUNK, NCHUNK), lambda i: (i, 0, 0)),
        ],
        out_specs=[
            pl.BlockSpec((1, CHUNK, NCHUNK), lambda i: (i, 0, 0)),
            pl.BlockSpec((1, 1, 128), lambda i: (i, 0, 0)),
        ],
        out_shape=[
            jax.ShapeDtypeStruct((B, CHUNK, NCHUNK), jnp.float32),
            jax.ShapeDtypeStruct((B, 1, 128), jnp.float32),
        ],
        interpret=_INTERPRET,
    )(cls_pred, labT)

    negd = neg.transpose(0, 2, 1).reshape(B, N)
    stats_sc = jnp.stack([
        stats1[:, 0, 0], stats1[:, 0, 1], stats2[:, 0, 0],
        jnp.zeros((B,), jnp.float32),
    ])

    out = _sc_mine_call()(negd, stats_sc)
    return (out[0], out[1])


# speedup vs baseline: 1.3515x; 1.0004x over previous
import functools

import jax
import jax.numpy as jnp
from jax import lax
from jax.experimental import pallas as pl
from jax.experimental.pallas import tpu as pltpu
from jax.experimental.pallas import tpu_sc as plsc

N = 20000
B = 16
NOBJ = 32
C = 81
THRESHOLD = 0.5
NEG_POS = 3
AS, AL = 160, 125
CHUNK, NCHUNK = 800, 25

_INTERPRET = False


def _match_kernel(db_ref, boxes_ref, labels_ref, locs_ref, tcls_ref, stats_ref,
                  iou_ref):
    dcx = db_ref[0]
    dcy = db_ref[1]
    dw = db_ref[2]
    dh = db_ref[3]
    dx0 = dcx - dw / 2.0
    dy0 = dcy - dh / 2.0
    dx1 = dcx + dw / 2.0
    dy1 = dcy + dh / 2.0
    darea = (dx1 - dx0) * (dy1 - dy0)

    row_ids = jax.lax.broadcasted_iota(jnp.int32, (AS, AL), 0)
    col_ids = jax.lax.broadcasted_iota(jnp.int32, (AS, AL), 1)
    aidx = row_ids * AL + col_ids

    best = jnp.full((AS, AL), -1.0, jnp.float32)
    besti = jnp.zeros((AS, AL), jnp.int32)
    for j in range(NOBJ):
        bx0 = boxes_ref[0, 0, j]
        by0 = boxes_ref[0, 1, j]
        bx1 = boxes_ref[0, 2, j]
        by1 = boxes_ref[0, 3, j]
        barea = (bx1 - bx0) * (by1 - by0)
        ix = jnp.maximum(jnp.minimum(bx1, dx1) - jnp.maximum(bx0, dx0), 0.0)
        iy = jnp.maximum(jnp.minimum(by1, dy1) - jnp.maximum(by0, dy0), 0.0)
        inter = ix * iy
        union = jnp.maximum(barea + darea - inter, 1e-10)
        iou = inter / union
        upd = iou > best
        besti = jnp.where(upd, j, besti)
        best = jnp.where(upd, iou, best)
        iou_ref[j] = iou

    iou_all = iou_ref[...]
    rowmax = jnp.max(iou_all, axis=(1, 2), keepdims=True)
    cand = jnp.where(iou_all == rowmax, aidx[None], N)
    dbj3 = jnp.min(cand, axis=(1, 2), keepdims=True)
    jidx = jax.lax.broadcasted_iota(jnp.int32, (NOBJ, 1, 1), 0)
    fcand = jnp.where(aidx[None] == dbj3, jidx, -1)
    fj = jnp.max(fcand, axis=0)
    o = jnp.where(fj >= 0, fj, besti)
    ovl = jnp.where(fj >= 0, 1.0, best)

    lab = jnp.zeros((AS, AL), jnp.int32)
    gx0 = jnp.zeros((AS, AL), jnp.float32)
    gy0 = jnp.zeros((AS, AL), jnp.float32)
    gx1 = jnp.zeros((AS, AL), jnp.float32)
    gy1 = jnp.zeros((AS, AL), jnp.float32)
    for j in range(NOBJ):
        m = o == j
        lab = jnp.where(m, labels_ref[0, 0, j], lab)
        gx0 = jnp.where(m, boxes_ref[0, 0, j], gx0)
        gy0 = jnp.where(m, boxes_ref[0, 1, j], gy0)
        gx1 = jnp.where(m, boxes_ref[0, 2, j], gx1)
        gy1 = jnp.where(m, boxes_ref[0, 3, j], gy1)
    lab = jnp.where(ovl < THRESHOLD, 0, lab)
    tcls_ref[0] = lab

    pos = lab != 0
    n_pos = jnp.sum(pos.astype(jnp.float32))

    cxt = (gx0 + gx1) / 2.0
    cyt = (gy0 + gy1) / 2.0
    wt = gx1 - gx0
    ht = gy1 - gy0
    g0 = (cxt - dcx) / (dw / 10.0)
    g1 = (cyt - dcy) / (dh / 10.0)
    g2 = jnp.log(jnp.maximum(wt / dw, 1e-8)) * 5.0
    g3 = jnp.log(jnp.maximum(ht / dh, 1e-8)) * 5.0
    sl = jnp.zeros((AS, AL), jnp.float32)
    for c, g in enumerate((g0, g1, g2, g3)):
        ad = jnp.abs(locs_ref[0, c] - g)
        sl = sl + jnp.where(ad < 1.0, 0.5 * ad * ad, ad - 0.5)
    loc_num = jnp.sum(jnp.where(pos, sl, 0.0))

    lane = jax.lax.broadcasted_iota(jnp.int32, (1, 128), 1)
    stats_ref[0] = jnp.where(lane == 0, n_pos, jnp.where(lane == 1, loc_num, 0.0))


def _ce_kernel(cls_ref, lab_ref, neg_ref, stats_ref):
    cls_iota = jax.lax.broadcasted_iota(jnp.int32, (CHUNK, C), 1)
    acc = jnp.float32(0.0)
    for i in range(NCHUNK):
        x = cls_ref[0, pl.ds(i * CHUNK, CHUNK), :]
        lab = lab_ref[0, :, i:i + 1]
        s = jnp.sum(jnp.exp(x), axis=1, keepdims=True)
        tgt = jnp.sum(jnp.where(cls_iota == lab, x, 0.0), axis=1, keepdims=True)
        ce = jnp.log(s) - tgt
        posm = lab != 0
        neg_ref[0, :, i:i + 1] = jnp.where(posm, 0.0, ce)
        acc = acc + jnp.sum(jnp.where(posm, ce, 0.0))
    lane = jax.lax.broadcasted_iota(jnp.int32, (1, 128), 1)
    stats_ref[0] = jnp.where(lane == 0, acc, 0.0)


def _sc_gather16(x, idx):
    return lax.gather(
        x, idx[:, None],
        lax.GatherDimensionNumbers(offset_dims=(), collapsed_slice_dims=(0,),
                                   start_index_map=(0,)),
        (1,), mode=lax.GatherScatterMode.PROMISE_IN_BOUNDS)


def _sc_mine(neg_hbm, stats_hbm, out_hbm, data_v, stats_v, pub_v, all_v, shared):
    c = lax.axis_index("c")
    s = lax.axis_index("s")
    lane = lax.iota(jnp.int32, 16)
    nchunk = N // 16

    def vtree(x, op):
        for kk in (8, 4, 2, 1):
            x = op(x, _sc_gather16(x, lane ^ kk))
        return x

    @pl.when(c == 0)
    def _work():
        pltpu.sync_copy(stats_hbm, stats_v)
        pltpu.sync_copy(neg_hbm.at[s], data_v)
        npos16 = stats_v[0]
        npos_w = vtree(jnp.where(lane == s, npos16, 0.0), jnp.add)
        k = jnp.minimum(jnp.float32(NEG_POS) * npos_w, jnp.float32(N))

        def cmax(i, acc):
            for u in range(4):
                acc = jnp.maximum(acc, data_v[pl.ds(i * 64 + u * 16, 16)])
            return acc

        vmax = vtree(lax.fori_loop(0, nchunk // 4, cmax,
                                   jnp.zeros((16,), jnp.float32)), jnp.maximum)

        def count_ge(mid):
            def chunk(i, acc):
                for u in range(4):
                    acc = acc + jnp.where(
                        data_v[pl.ds(i * 64 + u * 16, 16)] >= mid, 1.0, 0.0)
                return acc
            return vtree(lax.fori_loop(0, nchunk // 4, chunk,
                                       jnp.zeros((16,), jnp.float32)), jnp.add)

        def step(_, carry):
            lo, hi = carry
            mid = 0.5 * (lo + hi)
            good = count_ge(mid) >= k
            return jnp.where(good, mid, lo), jnp.where(good, hi, mid)

        t, _ = lax.fori_loop(0, 31, step,
                             (jnp.zeros((16,), jnp.float32), vmax))

        def chunk2(i, carry):
            sa, ma = carry
            for u in range(4):
                v = data_v[pl.ds(i * 64 + u * 16, 16)]
                gt = v > t
                sa = sa + jnp.where(gt, v, 0.0)
                ma = ma + jnp.where(gt, 1.0, 0.0)
            return sa, ma

        sa, ma = lax.fori_loop(0, nchunk // 4, chunk2,
                               (jnp.zeros((16,), jnp.float32),
                                jnp.zeros((16,), jnp.float32)))
        m = vtree(ma, jnp.add)
        stot = vtree(sa, jnp.add)
        hard = jnp.where(k > 0, stot + (k - m) * t, 0.0)
        pub_v[...] = jnp.where(lane == s, hard, 0.0)
        pltpu.sync_copy(pub_v, shared.at[s])

    plsc.subcore_barrier()

    @pl.when((c == 0) & (s == 0))
    def _agg():
        pltpu.sync_copy(shared, all_v)
        h16 = jnp.zeros((16,), jnp.float32)
        for i in range(16):
            h16 = h16 + all_v[i]
        npos_tot = vtree(stats_v[0], jnp.add)
        loc_num = vtree(stats_v[1], jnp.add)
        conf_pos = vtree(stats_v[2], jnp.add)
        hard_tot = vtree(h16, jnp.add)
        loc_loss = loc_num / jnp.maximum(4.0 * npos_tot, 1.0)
        conf_loss = (hard_tot + conf_pos) / jnp.maximum(npos_tot, 1.0)
        pub_v[...] = jnp.where(lane == 0, loc_loss,
                               jnp.where(lane == 1, conf_loss, 0.0))
        pltpu.sync_copy(pub_v, out_hbm)


@functools.cache
def _sc_mine_call():
    return functools.partial(
        pl.kernel,
        out_type=jax.ShapeDtypeStruct((16,), jnp.float32),
        mesh=plsc.VectorSubcoreMesh(core_axis_name="c", subcore_axis_name="s"),
        scratch_types=[
            pltpu.VMEM((N,), jnp.float32),
            pltpu.VMEM((4, 16), jnp.float32),
            pltpu.VMEM((16,), jnp.float32),
            pltpu.VMEM((16, 16), jnp.float32),
            pltpu.HBM((16, 16), jnp.float32),
        ],
    )(_sc_mine)


@jax.jit
def kernel(locs_pred, cls_pred, boxes, labels, default_boxes):
    db3 = default_boxes.T.reshape(4, AS, AL)
    boxesT = boxes.transpose(0, 2, 1)
    labels3 = labels.reshape(B, 1, NOBJ)
    locsT = locs_pred.transpose(0, 2, 1).reshape(B, 4, AS, AL)

    tcls, stats1 = pl.pallas_call(
        _match_kernel,
        grid=(B,),
        in_specs=[
            pl.BlockSpec((4, AS, AL), lambda i: (0, 0, 0)),
            pl.BlockSpec((1, 4, NOBJ), lambda i: (i, 0, 0), memory_space=pltpu.SMEM),
            pl.BlockSpec((1, 1, NOBJ), lambda i: (i, 0, 0), memory_space=pltpu.SMEM),
            pl.BlockSpec((1, 4, AS, AL), lambda i: (i, 0, 0, 0)),
        ],
        out_specs=[
            pl.BlockSpec((1, AS, AL), lambda i: (i, 0, 0)),
            pl.BlockSpec((1, 1, 128), lambda i: (i, 0, 0)),
        ],
        out_shape=[
            jax.ShapeDtypeStruct((B, AS, AL), jnp.int32),
            jax.ShapeDtypeStruct((B, 1, 128), jnp.float32),
        ],
        scratch_shapes=[pltpu.VMEM((NOBJ, AS, AL), jnp.float32)],
        interpret=_INTERPRET,
    )(db3, boxesT, labels3, locsT)

    labT = tcls.reshape(B, NCHUNK, CHUNK).transpose(0, 2, 1)

    neg, stats2 = pl.pallas_call(
        _ce_kernel,
        grid=(B,),
        in_specs=[
            pl.BlockSpec((1, N, C), lambda i: (i, 0, 0)),
            pl.BlockSpec((1, CHUNK, NCHUNK), lambda i: (i, 0, 0)),
        ],
        out_specs=[
            pl.BlockSpec((1, CHUNK, NCHUNK), lambda i: (i, 0, 0)),
            pl.BlockSpec((1, 1, 128), lambda i: (i, 0, 0)),
        ],
        out_shape=[
            jax.ShapeDtypeStruct((B, CHUNK, NCHUNK), jnp.float32),
            jax.ShapeDtypeStruct((B, 1, 128), jnp.float32),
        ],
        interpret=_INTERPRET,
    )(cls_pred, labT)

    negd = neg.reshape(B, N)
    stats_sc = jnp.stack([
        stats1[:, 0, 0], stats1[:, 0, 1], stats2[:, 0, 0],
        jnp.zeros((B,), jnp.float32),
    ])

    out = _sc_mine_call()(negd, stats_sc)
    return (out[0], out[1])
